# baseline (speedup 1.0000x reference)
import functools
import math

import jax
import jax.numpy as jnp
from jax import lax
from jax.experimental import pallas as pl
from jax.experimental.pallas import tpu as pltpu
from jax.experimental.pallas import tpu_sc as plsc

D = 32
SCALE = math.sqrt(D)

_info = plsc.get_sparse_core_info()
NC, NS, L = _info.num_cores, _info.num_subcores, _info.num_lanes
NW = NC * NS

BATCH = 4096
SEQ = 200
BC = BATCH // NW
TILE = 8 * BC
YW = D // 8 * TILE
NBUF = 4
Y5_SHAPE = (SEQ, D // 8, NW, 8, BC)
Y_FLAT = SEQ * D // 8 * NW * 8 * BC


def _tec_transform(rows_ref, ybuf_ref):
    d = lax.iota(jnp.int32, L)
    c0 = (d >> 3) * TILE + (d & 7) * BC
    c1 = c0 + 2 * TILE

    def row_step(bb, _):
        r0 = rows_ref[bb, pl.ds(0, L)]
        plsc.store_scatter(ybuf_ref, [c0 + bb], r0 * SCALE)
        r1 = rows_ref[bb, pl.ds(L, L)]
        plsc.store_scatter(ybuf_ref, [c1 + bb], r1 * SCALE)
        return 0

    lax.fori_loop(0, BC, row_step, 0, unroll=8)


def _emb_body(xT_hbm, lut_hbm, y_hbm, xblk, rows, ybufs, gsems, wsems):
    wid = lax.axis_index("s") * NC + lax.axis_index("c")
    b0 = wid * BC

    pltpu.sync_copy(xT_hbm.at[:, pl.ds(b0, BC)], xblk)

    def gather_start(t, slot):
        pltpu.async_copy(lut_hbm.at[xblk.at[t]], rows[slot], gsems[slot])

    def gather_wait(t, slot):
        pltpu.make_async_copy(lut_hbm.at[xblk.at[t]], rows[slot], gsems[slot]).wait()

    def seg(t, g):
        off = ((t * (D // 8) + g) * NW + wid) * TILE
        return y_hbm.at[pl.ds(off, TILE)]

    def write_start(t, slot):
        for g in range(D // 8):
            pltpu.async_copy(ybufs[slot].at[pl.ds(g * TILE, TILE)], seg(t, g), wsems[slot])

    def write_wait(t, slot):
        for g in range(D // 8):
            pltpu.make_async_copy(
                ybufs[slot].at[pl.ds(g * TILE, TILE)], seg(t, g), wsems[slot]
            ).wait()

    for s in range(NBUF):
        gather_start(s, s)

    def step(k, _):
        for s in range(NBUF):
            t = NBUF * k + s
            gather_wait(t, s)

            @pl.when(k > 0)
            def _():
                write_wait(t - NBUF, s)

            _tec_transform(rows[s], ybufs[s])
            write_start(t, s)

            @pl.when(k < SEQ // NBUF - 1)
            def _():
                gather_start(t + NBUF, s)

        return 0

    lax.fori_loop(0, SEQ // NBUF, step, 0)

    for s in range(NBUF):
        write_wait(SEQ - NBUF + s, s)


@jax.jit
def _emb(xT, lut):
    mesh = plsc.VectorSubcoreMesh(core_axis_name="c", subcore_axis_name="s")
    f = functools.partial(
        pl.kernel,
        mesh=mesh,
        out_type=jax.ShapeDtypeStruct((Y_FLAT,), jnp.float32),
        scratch_types=[
            pltpu.VMEM((SEQ, BC), jnp.int32),
            [pltpu.VMEM((BC, D), jnp.float32) for _ in range(NBUF)],
            [pltpu.VMEM((D // 8 * TILE,), jnp.float32) for _ in range(NBUF)],
            [pltpu.SemaphoreType.DMA for _ in range(NBUF)],
            [pltpu.SemaphoreType.DMA for _ in range(NBUF)],
        ],
        compiler_params=pltpu.CompilerParams(
            use_tc_tiling_on_sc=False, needs_layout_passes=False
        ),
    )(_emb_body)
    return f(xT, lut)


def kernel(x, lut):
    xT = x.T.astype(jnp.int32)
    y5 = _emb(xT, lut).reshape(Y5_SHAPE)
    return jnp.transpose(y5, (2, 4, 0, 1, 3)).reshape(BATCH, SEQ, D)


# speedup vs baseline: 1.3797x; 1.3797x over previous
import functools
import math

import jax
import jax.numpy as jnp
from jax import lax
from jax.experimental import pallas as pl
from jax.experimental.pallas import tpu as pltpu
from jax.experimental.pallas import tpu_sc as plsc

D = 32
SCALE = math.sqrt(D)

_info = plsc.get_sparse_core_info()
NC, NS, L = _info.num_cores, _info.num_subcores, _info.num_lanes
NW = NC * NS

BATCH = 4096
SEQ = 200
BC = BATCH // NW
TS = 4
NSTEP = SEQ // TS
PITCH = BC + 1
NBUF = 3
Y5_SHAPE = (SEQ, D // 8, NW, 8, BC)


def _tec_transform(rows_ref, ybuf_ref):
    d = lax.iota(jnp.int32, L)
    i_dgrp0 = d >> 3
    i_dd = d & 7
    i_dgrp1 = i_dgrp0 + 2
    zero = jnp.full((L,), 0, jnp.int32)

    for tl in range(TS):
        i_tl = jnp.full((L,), tl, jnp.int32)

        def row_step(bb, _):
            i_bb = zero + bb
            r0 = rows_ref[tl * BC + bb, pl.ds(0, L)]
            plsc.store_scatter(ybuf_ref, [i_tl, i_dgrp0, i_dd, i_bb], r0 * SCALE)
            r1 = rows_ref[tl * BC + bb, pl.ds(L, L)]
            plsc.store_scatter(ybuf_ref, [i_tl, i_dgrp1, i_dd, i_bb], r1 * SCALE)
            return 0

        lax.fori_loop(0, BC, row_step, 0, unroll=8)


def _emb_body(xg_hbm, lut_hbm, y5_hbm, xblk, rows, ybufs, gsems, wsems):
    wid = lax.axis_index("s") * NC + lax.axis_index("c")

    pltpu.sync_copy(xg_hbm.at[wid], xblk)

    def gather_start(k, slot):
        pltpu.async_copy(lut_hbm.at[xblk.at[pl.ds(k * TS * BC, TS * BC)]],
                         rows[slot], gsems[slot])

    def gather_wait(k, slot):
        pltpu.make_async_copy(lut_hbm.at[xblk.at[pl.ds(k * TS * BC, TS * BC)]],
                              rows[slot], gsems[slot]).wait()

    def wpair(k, slot):
        src = ybufs[slot].at[:, :, :, pl.ds(0, BC)]
        dst = y5_hbm.at[pl.ds(k * TS, TS), :, wid]
        return src, dst

    def write_start(k, slot):
        src, dst = wpair(k, slot)
        pltpu.async_copy(src, dst, wsems[slot])

    def write_wait(k, slot):
        src, dst = wpair(k, slot)
        pltpu.make_async_copy(src, dst, wsems[slot]).wait()

    for s in range(NBUF):
        gather_start(s, s)

    def step(j, _):
        for s in range(NBUF):
            k = NBUF * j + s
            gather_wait(k, s)

            @pl.when(k >= NBUF)
            def _():
                write_wait(k - NBUF, s)

            _tec_transform(rows[s], ybufs[s])
            write_start(k, s)

            @pl.when(k < NSTEP - NBUF)
            def _():
                gather_start(k + NBUF, s)

        return 0

    lax.fori_loop(0, NSTEP // NBUF, step, 0)
    for k in range(NSTEP // NBUF * NBUF, NSTEP):
        s = k % NBUF
        gather_wait(k, s)
        write_wait(k - NBUF, s)
        _tec_transform(rows[s], ybufs[s])
        write_start(k, s)

    for k in range(NSTEP - NBUF, NSTEP):
        write_wait(k, k % NBUF)


@jax.jit
def _emb(xg, lut):
    mesh = plsc.VectorSubcoreMesh(core_axis_name="c", subcore_axis_name="s")
    f = functools.partial(
        pl.kernel,
        mesh=mesh,
        out_type=jax.ShapeDtypeStruct(Y5_SHAPE, jnp.float32),
        scratch_types=[
            pltpu.VMEM((SEQ * BC,), jnp.int32),
            [pltpu.VMEM((TS * BC, D), jnp.float32) for _ in range(NBUF)],
            [pltpu.VMEM((TS, D // 8, 8, PITCH), jnp.float32) for _ in range(NBUF)],
            [pltpu.SemaphoreType.DMA for _ in range(NBUF)],
            [pltpu.SemaphoreType.DMA for _ in range(NBUF)],
        ],
        compiler_params=pltpu.CompilerParams(
            use_tc_tiling_on_sc=False, needs_layout_passes=False
        ),
    )(_emb_body)
    return f(xg, lut)


def kernel(x, lut):
    xg = (
        x.T.astype(jnp.int32)
        .reshape(SEQ, NW, BC)
        .transpose(1, 0, 2)
        .reshape(NW, SEQ * BC)
    )
    y5 = _emb(xg, lut)
    return jnp.transpose(y5, (2, 4, 0, 1, 3)).reshape(BATCH, SEQ, D)
